# Initial kernel scaffold; baseline (speedup 1.0000x reference)
import jax
import jax.numpy as jnp
from jax.experimental import pallas as pl


def kernel(X, table):
    raise NotImplementedError("write your pallas kernel here")


# speedup vs baseline: 1.8759x; 1.8759x over previous
import functools

import jax
import jax.numpy as jnp
from jax import lax
from jax.experimental import pallas as pl
from jax.experimental.pallas import tpu as pltpu
from jax.experimental.pallas import tpu_sc as plsc

NC = 2
NS = 16
NW = NC * NS
CHUNK = 128
NBUF = 8


@functools.lru_cache(maxsize=None)
def _build(V, D, B):
    assert B % (NW * CHUNK) == 0
    rows_per_w = B // NW
    n_chunks = rows_per_w // CHUNK
    n_rounds = n_chunks // NBUF
    assert n_chunks % NBUF == 0

    mesh = plsc.VectorSubcoreMesh(
        core_axis_name="c", subcore_axis_name="s",
        num_cores=NC, num_subcores=NS)

    @functools.partial(
        pl.kernel,
        out_type=jax.ShapeDtypeStruct((B, D), jnp.float32),
        mesh=mesh,
        scratch_types=[
            pltpu.VMEM((n_chunks, CHUNK), jnp.int32),
            *[pltpu.VMEM((CHUNK, D), jnp.float32) for _ in range(NBUF)],
            *[pltpu.SemaphoreType.DMA for _ in range(2 * NBUF)],
        ],
        compiler_params=pltpu.CompilerParams(use_tc_tiling_on_sc=False),
    )
    def gather_kernel(idx_hbm, table_hbm, out_hbm, idx_v, *rest):
        bufs = rest[:NBUF]
        gsems = rest[NBUF:2 * NBUF]
        wsems = rest[2 * NBUF:]
        wid = lax.axis_index("s") * NC + lax.axis_index("c")
        chunk0 = wid * n_chunks

        pltpu.sync_copy(idx_hbm.at[pl.ds(chunk0, n_chunks)], idx_v)

        for b in range(NBUF):
            pltpu.async_copy(table_hbm.at[idx_v.at[b]], bufs[b], gsems[b])

        @pl.loop(0, n_rounds)
        def _round(r):
            base = r * NBUF
            for b in range(NBUF):
                c = base + b
                pltpu.make_async_copy(
                    table_hbm.at[idx_v.at[c]], bufs[b], gsems[b]).wait()
                out_slice = out_hbm.at[pl.ds((chunk0 + c) * CHUNK, CHUNK)]
                pltpu.async_copy(bufs[b], out_slice, wsems[b])
                c2 = c + NBUF

                @pl.when(c2 < n_chunks)
                def _refill():
                    pltpu.make_async_copy(bufs[b], out_slice, wsems[b]).wait()
                    pltpu.async_copy(
                        table_hbm.at[idx_v.at[c2]], bufs[b], gsems[b])

        for b in range(NBUF):
            c = (n_rounds - 1) * NBUF + b
            out_slice = out_hbm.at[pl.ds((chunk0 + c) * CHUNK, CHUNK)]
            pltpu.make_async_copy(bufs[b], out_slice, wsems[b]).wait()

    return gather_kernel


def kernel(X, table):
    B = X.shape[0] * X.shape[1]
    D = table.shape[1]
    idx = X.reshape(B // CHUNK, CHUNK).astype(jnp.int32)
    out = _build(table.shape[0], D, B)(idx, table)
    return out.reshape(X.shape[0], X.shape[1], D)
